# baseline (speedup 1.0000x reference)
import functools

import jax
import jax.numpy as jnp
from jax import lax
from jax.experimental import pallas as pl
from jax.experimental.pallas import tpu as pltpu
from jax.experimental.pallas import tpu_sc as plsc

_SB = 4


def _make_sc_embed(B, S, V, D, NC, NS):
    NW = NC * NS
    BT = B // NW
    ST = S // 8
    n_batches = S // _SB
    NI = n_batches // 2
    CH = _SB * BT
    DT, DI = D // 8, 8
    mesh = plsc.VectorSubcoreMesh(core_axis_name="c", subcore_axis_name="s")

    @functools.partial(
        pl.kernel,
        mesh=mesh,
        out_type=jax.ShapeDtypeStruct((S, DT, NW, DI, BT), jnp.float32),
        compiler_params=pltpu.CompilerParams(
            use_tc_tiling_on_sc=False, needs_layout_passes=False
        ),
        scratch_types=[
            pltpu.VMEM((ST, 8 * BT), jnp.int32),
            pltpu.VMEM((CH, D), jnp.float32),
            pltpu.VMEM((CH, D), jnp.float32),
            pltpu.VMEM((DT, _SB, DI, BT), jnp.float32),
            pltpu.VMEM((DT, _SB, DI, BT), jnp.float32),
            pltpu.SemaphoreType.DMA,
            pltpu.SemaphoreType.DMA,
            pltpu.SemaphoreType.DMA,
            pltpu.SemaphoreType.DMA,
        ],
    )
    def emb(idx_hbm, table_hbm, out_hbm, idx_v, rows0, rows1, tv0, tv1,
            g0, g1, o0, o1):
        wid = lax.axis_index("s") * NC + lax.axis_index("c")
        pltpu.sync_copy(idx_hbm.at[:, wid], idx_v)

        rows = [rows0, rows1]
        tv = [tv0, tv1]
        gsem = [g0, g1]
        osem = [o0, o1]

        iota = lax.iota(jnp.int32, 16)

        def idx_ref(p, st):
            return idx_v.at[st, pl.ds(p * CH, CH)]

        def start_gather(p, st):
            return pltpu.async_copy(
                table_hbm.at[idx_ref(p, st)], rows[p], gsem[p]
            )

        def wait_gather(p, st):
            pltpu.make_async_copy(
                table_hbm.at[idx_ref(p, st)], rows[p], gsem[p]
            ).wait()

        def permute(p):
            rp = rows[p]
            tp = tv[p]

            @plsc.parallel_loop(0, D, unroll=8)
            def _(d0):
                dcol = (d0 + iota) & (D - 1)
                dtv = dcol >> 3
                div = dcol & 7
                for sb in range(_SB):
                    sbv = jnp.full((16,), sb, jnp.int32)
                    for b0 in range(0, BT, 16):
                        val = plsc.load_gather(
                            rp, [iota + (sb * BT + b0), dcol]
                        )
                        plsc.store_scatter(tp, [dtv, sbv, div, iota + b0], val)

        def start_out(p, k):
            s0 = k * _SB
            for dt in range(DT):
                pltpu.async_copy(
                    tv[p].at[dt], out_hbm.at[pl.ds(s0, _SB), dt, wid], osem[p]
                )

        def wait_out(p):
            for _ in range(DT):
                pltpu.make_async_copy(
                    tv[p].at[0], out_hbm.at[pl.ds(0, _SB), 0, wid], osem[p]
                ).wait()

        start_gather(0, 0)
        start_gather(1, 0)

        def step(i, p):
            k = 2 * i + p

            @pl.when(i >= 1)
            def _():
                wait_out(p)

            wait_gather(p, i)
            permute(p)
            start_out(p, k)

            @pl.when(i < NI - 1)
            def _():
                start_gather(p, i + 1)

        def body(i, carry):
            step(i, 0)
            step(i, 1)
            return carry

        lax.fori_loop(0, NI, body, 0)
        wait_out(0)
        wait_out(1)

    return emb


def kernel(token_ids, table):
    B, S = token_ids.shape
    V, D = table.shape
    idx4 = (
        token_ids.astype(jnp.int32)
        .T.reshape(S // 8, 8, B // 128, 128)
        .transpose(0, 2, 1, 3)
        .reshape(S // 8, B // 128, 8 * 128)
    )
    info = plsc.get_sparse_core_info()
    NC, NS = info.num_cores, info.num_subcores
    emb = _make_sc_embed(B, S, V, D, NC, NS)
    out5 = emb(idx4, table)
    return out5.transpose(2, 4, 0, 1, 3).reshape(B, S, D)


# speedup vs baseline: 1.0124x; 1.0124x over previous
import functools

import jax
import jax.numpy as jnp
from jax import lax
from jax.experimental import pallas as pl
from jax.experimental.pallas import tpu as pltpu
from jax.experimental.pallas import tpu_sc as plsc

_SB = 4


def _make_sc_embed(B, S, V, D, NC, NS):
    NW = NC * NS
    BT = B // NW
    ST = S // 8
    n_batches = S // _SB
    NI = n_batches // 2
    CH = _SB * BT
    DT, DI = D // 8, 8
    mesh = plsc.VectorSubcoreMesh(core_axis_name="c", subcore_axis_name="s")

    @functools.partial(
        pl.kernel,
        mesh=mesh,
        out_type=jax.ShapeDtypeStruct((S, DT, NW, DI, BT), jnp.float32),
        compiler_params=pltpu.CompilerParams(
            use_tc_tiling_on_sc=False, needs_layout_passes=False
        ),
        scratch_types=[
            pltpu.VMEM((ST, 8 * BT), jnp.int32),
            pltpu.VMEM((CH, D), jnp.float32),
            pltpu.VMEM((CH, D), jnp.float32),
            pltpu.VMEM((DT, _SB, DI, BT), jnp.float32),
            pltpu.VMEM((DT, _SB, DI, BT), jnp.float32),
            pltpu.SemaphoreType.DMA,
            pltpu.SemaphoreType.DMA,
            pltpu.SemaphoreType.DMA,
            pltpu.SemaphoreType.DMA,
        ],
    )
    def emb(idx_hbm, table_hbm, out_hbm, idx_v, rows0, rows1, tv0, tv1,
            g0, g1, o0, o1):
        wid = lax.axis_index("s") * NC + lax.axis_index("c")
        pltpu.sync_copy(idx_hbm.at[:, wid], idx_v)

        rows = [rows0, rows1]
        tv = [tv0, tv1]
        gsem = [g0, g1]
        osem = [o0, o1]

        iota = lax.iota(jnp.int32, 16)

        def idx_ref(p, st):
            return idx_v.at[st, pl.ds(p * CH, CH)]

        def start_gather(p, st):
            return pltpu.async_copy(
                table_hbm.at[idx_ref(p, st)], rows[p], gsem[p]
            )

        def wait_gather(p, st):
            pltpu.make_async_copy(
                table_hbm.at[idx_ref(p, st)], rows[p], gsem[p]
            ).wait()

        def permute(p):
            rp = rows[p]
            tp = tv[p]

            @plsc.parallel_loop(0, D, unroll=4)
            def _(d0):
                dcol = (d0 + iota) & (D - 1)
                dtv = dcol >> 3
                div = dcol & 7
                for sb in range(_SB):
                    sbv = jnp.full((16,), sb, jnp.int32)
                    for b0 in range(0, BT, 16):
                        val = plsc.load_gather(
                            rp, [iota + (sb * BT + b0), dcol]
                        )
                        plsc.store_scatter(tp, [dtv, sbv, div, iota + b0], val)

        def start_out(p, k):
            s0 = k * _SB
            for dt in range(DT):
                pltpu.async_copy(
                    tv[p].at[dt], out_hbm.at[pl.ds(s0, _SB), dt, wid], osem[p]
                )

        def wait_out(p):
            for _ in range(DT):
                pltpu.make_async_copy(
                    tv[p].at[0], out_hbm.at[pl.ds(0, _SB), 0, wid], osem[p]
                ).wait()

        start_gather(0, 0)
        start_gather(1, 0)

        def step(i, p):
            k = 2 * i + p

            @pl.when(i >= 1)
            def _():
                wait_out(p)

            wait_gather(p, i)
            permute(p)
            start_out(p, k)

            @pl.when(i < NI - 1)
            def _():
                start_gather(p, i + 1)

        def body(i, carry):
            step(i, 0)
            step(i, 1)
            return carry

        lax.fori_loop(0, NI, body, 0)
        wait_out(0)
        wait_out(1)

    return emb


def kernel(token_ids, table):
    B, S = token_ids.shape
    V, D = table.shape
    idx4 = (
        token_ids.astype(jnp.int32)
        .T.reshape(S // 8, 8, B // 128, 128)
        .transpose(0, 2, 1, 3)
        .reshape(S // 8, B // 128, 8 * 128)
    )
    info = plsc.get_sparse_core_info()
    NC, NS = info.num_cores, info.num_subcores
    emb = _make_sc_embed(B, S, V, D, NC, NS)
    out5 = emb(idx4, table)
    return out5.transpose(2, 4, 0, 1, 3).reshape(B, S, D)
